# baseline (speedup 1.0000x reference)
import functools
import jax
import jax.numpy as jnp
from jax import lax
from jax.experimental import pallas as pl
from jax.experimental.pallas import tpu as pltpu
from jax.experimental.pallas import tpu_sc as plsc

N, E, F, R, P = 10000, 320000, 128, 16, 2
NC, NS = 2, 16
NW = NC * NS
EPW = E // NW
CHUNK = 80
NCHUNK = EPW // CHUNK
SUPER = 25
NSUPER = NCHUNK // SUPER
RPT = 624
TAIL = N - NS * RPT
NBLK = 1000
EBLK = 1000


def _node_fwd_body(emb_ref, wi_ref, bi_ref, wj_ref, bj_ref, hi0h_ref, hjp_ref):
    emb = emb_ref[...]
    hi = jnp.maximum(
        jnp.dot(emb, wi_ref[...], preferred_element_type=jnp.float32)
        + bi_ref[...], 0.0)
    hi0h_ref[...] = hi * 0.5
    hjp_ref[...] = jnp.maximum(
        jnp.dot(emb, wj_ref[...], preferred_element_type=jnp.float32)
        + bj_ref[...], 0.0)


def _node_fwd(emb, Wi, bi, Wj, bj):
    grid = (N // NBLK,)
    row = pl.BlockSpec((NBLK, F), lambda i: (i, 0))
    full = pl.BlockSpec((F, F), lambda i: (0, 0))
    vec = pl.BlockSpec((1, F), lambda i: (0, 0))
    return pl.pallas_call(
        _node_fwd_body,
        grid=grid,
        in_specs=[row, full, vec, full, vec],
        out_specs=[row, row],
        out_shape=[jax.ShapeDtypeStruct((N, F), jnp.float32),
                   jax.ShapeDtypeStruct((N, F), jnp.float32)],
    )(emb, Wi, bi.reshape(1, F), Wj, bj.reshape(1, F))


def _g_body(f_ref, wexp_ref, g_ref):
    g_ref[...] = jnp.dot(f_ref[...], wexp_ref[...],
                         preferred_element_type=jnp.float32)


def _edge_gate(f_ij, Wg):
    pack = F // R
    f_rs = f_ij.reshape(E // pack, F)
    wexp = jnp.kron(jnp.eye(pack, dtype=jnp.float32), Wg)
    grid = ((E // pack) // EBLK,)
    out = pl.pallas_call(
        _g_body,
        grid=grid,
        in_specs=[pl.BlockSpec((EBLK, F), lambda i: (i, 0)),
                  pl.BlockSpec((F, pack * F), lambda i: (0, 0))],
        out_specs=pl.BlockSpec((EBLK, pack * F), lambda i: (i, 0)),
        out_shape=jax.ShapeDtypeStruct((E // pack, pack * F), jnp.float32),
    )(f_rs, wexp)
    return out.reshape(E, F)


def _sc_edge_body(hi0h, hjp, g, idxj, idxi, out,
                  acc, idxj_v, idxi_v, rows_v, g_v, sem_r, sem_g):
    c = lax.axis_index("c")
    s = lax.axis_index("s")
    wid = c * NS + s

    pltpu.sync_copy(hi0h.at[pl.ds(s * RPT, RPT)], acc.at[pl.ds(s * RPT, RPT)])

    @pl.when(s == NS - 1)
    def _():
        pltpu.sync_copy(hi0h.at[pl.ds(NS * RPT, TAIL)],
                        acc.at[pl.ds(NS * RPT, TAIL)])

    plsc.subcore_barrier()

    def issue(t, v, buf):
        pltpu.async_copy(hjp.at[idxj_v.at[v]], rows_v.at[buf], sem_r.at[buf])
        pltpu.async_copy(g.at[pl.ds((wid * NCHUNK + t) * CHUNK, CHUNK)],
                         g_v.at[buf], sem_g.at[buf])

    def superchunk(u, carry):
        pltpu.sync_copy(idxj.at[wid, u], idxj_v)
        pltpu.sync_copy(idxi.at[wid, u], idxi_v)
        t0 = u * SUPER
        issue(t0, 0, lax.rem(t0, 2))

        def chunk(v, cc):
            t = t0 + v
            par = lax.rem(t, 2)

            @pl.when(v < SUPER - 1)
            def _():
                issue(t + 1, v + 1, 1 - par)

            pltpu.make_async_copy(hjp.at[idxj_v.at[v]], rows_v.at[par],
                                  sem_r.at[par]).wait()
            pltpu.make_async_copy(g.at[pl.ds(0, CHUNK)], g_v.at[par],
                                  sem_g.at[par]).wait()

            def mrow(r, c2):
                for k in range(F // 16):
                    sl = pl.ds(k * 16, 16)
                    rows_v[par, r, sl] = rows_v[par, r, sl] * g_v[par, r, sl]
                return c2
            lax.fori_loop(0, CHUNK, mrow, 0)

            pltpu.sync_copy(rows_v.at[par], acc.at[idxi_v.at[v]], add=True)
            return cc

        lax.fori_loop(0, SUPER, chunk, 0)
        return carry

    lax.fori_loop(0, NSUPER, superchunk, 0)
    plsc.subcore_barrier()

    pltpu.sync_copy(acc.at[pl.ds(s * RPT, RPT)],
                    out.at[c, pl.ds(s * RPT, RPT)])

    @pl.when(s == NS - 1)
    def _():
        pltpu.sync_copy(acc.at[pl.ds(NS * RPT, TAIL)],
                        out.at[c, pl.ds(NS * RPT, TAIL)])


def _sc_edge(hi0h, hjp, g, idx_j, idx_i):
    mesh = plsc.VectorSubcoreMesh(core_axis_name="c", subcore_axis_name="s")
    kern = pl.kernel(
        _sc_edge_body,
        out_type=jax.ShapeDtypeStruct((NC, N, F), jnp.float32),
        mesh=mesh,
        scratch_types=[
            pltpu.VMEM_SHARED((N, F), jnp.float32),
            pltpu.VMEM((SUPER, CHUNK), jnp.int32),
            pltpu.VMEM((SUPER, CHUNK), jnp.int32),
            pltpu.VMEM((2, CHUNK, F), jnp.float32),
            pltpu.VMEM((2, CHUNK, F), jnp.float32),
            pltpu.SemaphoreType.DMA((2,)),
            pltpu.SemaphoreType.DMA((2,)),
        ],
    )
    return kern(hi0h, hjp, g,
                idx_j.reshape(NW, NSUPER, SUPER, CHUNK),
                idx_i.reshape(NW, NSUPER, SUPER, CHUNK))


def _tail_body(emb_ref, a0_ref, a1_ref,
               iw10, ib10, iw20, ib20, iw11, ib11, iw21, ib21,
               wv, bv_, gate_, ow1, ob1, ow2, ob2, woutp, boutp,
               upd_ref, predp_ref):
    def res(x, w1, b1, w2, b2):
        t = jnp.maximum(x, 0.0)
        u = jnp.maximum(
            jnp.dot(t, w1[...], preferred_element_type=jnp.float32)
            + b1[...], 0.0)
        return x + jnp.dot(u, w2[...], preferred_element_type=jnp.float32) \
            + b2[...]

    h = a0_ref[0] + a1_ref[0]
    h = res(h, iw10, ib10, iw20, ib20)
    h = res(h, iw11, ib11, iw21, ib21)
    upd = gate_[...] * emb_ref[...] + jnp.dot(
        jnp.maximum(h, 0.0), wv[...],
        preferred_element_type=jnp.float32) + bv_[...]
    upd_ref[...] = upd
    y = res(upd, ow1, ob1, ow2, ob2)
    predp_ref[...] = jnp.dot(y, woutp[...],
                             preferred_element_type=jnp.float32) + boutp[...]


def _tail(emb, acc, ir0_W1, ir0_b1, ir0_W2, ir0_b2,
          ir1_W1, ir1_b1, ir1_W2, ir1_b2, Wv, bv, gate,
          or0_W1, or0_b1, or0_W2, or0_b2, Wout, bout):
    woutp = jnp.zeros((F, F), jnp.float32).at[:, :P].set(Wout)
    boutp = jnp.zeros((F,), jnp.float32).at[:P].set(bout)
    grid = (N // NBLK,)
    row = pl.BlockSpec((NBLK, F), lambda i: (i, 0))
    full = pl.BlockSpec((F, F), lambda i: (0, 0))
    vec = pl.BlockSpec((1, F), lambda i: (0, 0))
    a0s = pl.BlockSpec((1, NBLK, F), lambda i: (0, i, 0))
    a1s = pl.BlockSpec((1, NBLK, F), lambda i: (1, i, 0))
    upd, predp = pl.pallas_call(
        _tail_body,
        grid=grid,
        in_specs=[row, a0s, a1s,
                  full, vec, full, vec, full, vec, full, vec,
                  full, vec, vec, full, vec, full, vec, full, vec],
        out_specs=[row, row],
        out_shape=[jax.ShapeDtypeStruct((N, F), jnp.float32),
                   jax.ShapeDtypeStruct((N, F), jnp.float32)],
    )(emb, acc, acc,
      ir0_W1, ir0_b1.reshape(1, F), ir0_W2, ir0_b2.reshape(1, F),
      ir1_W1, ir1_b1.reshape(1, F), ir1_W2, ir1_b2.reshape(1, F),
      Wv, bv.reshape(1, F), gate.reshape(1, F),
      or0_W1, or0_b1.reshape(1, F), or0_W2, or0_b2.reshape(1, F),
      woutp, boutp.reshape(1, F))
    return predp[:, :P], upd


def kernel(atomic_embedding, pair_indices, f_ij, Wi, bi, Wj, bj, Wg, Wv, bv,
           gate, ir0_W1, ir0_b1, ir0_W2, ir0_b2, ir1_W1, ir1_b1, ir1_W2,
           ir1_b2, or0_W1, or0_b1, or0_W2, or0_b2, Wout, bout):
    idx_i = pair_indices[0].astype(jnp.int32)
    idx_j = pair_indices[1].astype(jnp.int32)
    hi0h, hjp = _node_fwd(atomic_embedding, Wi, bi, Wj, bj)
    g = _edge_gate(f_ij, Wg)
    acc = _sc_edge(hi0h, hjp, g, idx_j, idx_i)
    prediction, updated = _tail(
        atomic_embedding, acc,
        ir0_W1, ir0_b1, ir0_W2, ir0_b2, ir1_W1, ir1_b1, ir1_W2, ir1_b2,
        Wv, bv, gate, or0_W1, or0_b1, or0_W2, or0_b2, Wout, bout)
    return (prediction, updated)


# speedup vs baseline: 1.5670x; 1.5670x over previous
import functools
import jax
import jax.numpy as jnp
from jax import lax
from jax.experimental import pallas as pl
from jax.experimental.pallas import tpu as pltpu
from jax.experimental.pallas import tpu_sc as plsc

N, E, F, R, P = 10000, 320000, 128, 16, 2
NC, NS = 2, 16
NW = NC * NS
EPW = E // NW
CHUNK = 40
NCHUNK = EPW // CHUNK
NBUF = 4
RPT = 624
TAIL = N - NS * RPT
NBLK = 1000
EBLK = 1000


def _node_fwd_body(emb_ref, wi_ref, bi_ref, wj_ref, bj_ref, hi0h_ref, hjp_ref):
    emb = emb_ref[...]
    hi = jnp.maximum(
        jnp.dot(emb, wi_ref[...], preferred_element_type=jnp.float32)
        + bi_ref[...], 0.0)
    hi0h_ref[...] = hi * 0.5
    hjp_ref[...] = jnp.maximum(
        jnp.dot(emb, wj_ref[...], preferred_element_type=jnp.float32)
        + bj_ref[...], 0.0)


def _node_fwd(emb, Wi, bi, Wj, bj):
    grid = (N // NBLK,)
    row = pl.BlockSpec((NBLK, F), lambda i: (i, 0))
    full = pl.BlockSpec((F, F), lambda i: (0, 0))
    vec = pl.BlockSpec((1, F), lambda i: (0, 0))
    return pl.pallas_call(
        _node_fwd_body,
        grid=grid,
        in_specs=[row, full, vec, full, vec],
        out_specs=[row, row],
        out_shape=[jax.ShapeDtypeStruct((N, F), jnp.float32),
                   jax.ShapeDtypeStruct((N, F), jnp.float32)],
    )(emb, Wi, bi.reshape(1, F), Wj, bj.reshape(1, F))


def _g_body(f_ref, wexp_ref, g_ref):
    g_ref[...] = jnp.dot(f_ref[...], wexp_ref[...],
                         preferred_element_type=jnp.float32)


def _edge_gate(f_ij, Wg):
    pack = F // R
    f_rs = f_ij.reshape(E // pack, F)
    wexp = jnp.kron(jnp.eye(pack, dtype=jnp.float32), Wg)
    grid = ((E // pack) // EBLK,)
    out = pl.pallas_call(
        _g_body,
        grid=grid,
        in_specs=[pl.BlockSpec((EBLK, F), lambda i: (i, 0)),
                  pl.BlockSpec((F, pack * F), lambda i: (0, 0))],
        out_specs=pl.BlockSpec((EBLK, pack * F), lambda i: (i, 0)),
        out_shape=jax.ShapeDtypeStruct((E // pack, pack * F), jnp.float32),
    )(f_rs, wexp)
    return out.reshape(E, F)


def _sc_edge_body(hi0h, hjp, g, pair, out,
                  acc, idxj_v, idxi_v, rows_v, g_v, si, sr, sg, ss):
    c = lax.axis_index("c")
    s = lax.axis_index("s")
    wid = c * NS + s

    pltpu.sync_copy(hi0h.at[pl.ds(s * RPT, RPT)], acc.at[pl.ds(s * RPT, RPT)])

    @pl.when(s == NS - 1)
    def _():
        pltpu.sync_copy(hi0h.at[pl.ds(NS * RPT, TAIL)],
                        acc.at[pl.ds(NS * RPT, TAIL)])

    plsc.subcore_barrier()

    def issue_idx(t, b):
        pltpu.async_copy(pair.at[1, wid, t], idxj_v[b], si.at[b])
        pltpu.async_copy(pair.at[0, wid, t], idxi_v[b], si.at[b])

    def wait_idx(b):
        pltpu.make_async_copy(pair.at[1, wid, 0], idxj_v[b], si.at[b]).wait()
        pltpu.make_async_copy(pair.at[0, wid, 0], idxi_v[b], si.at[b]).wait()

    def issue_main(t, b):
        pltpu.async_copy(hjp.at[idxj_v[b]], rows_v[b], sr.at[b])
        pltpu.async_copy(g.at[pl.ds((wid * NCHUNK + t) * CHUNK, CHUNK)],
                         g_v[b], sg.at[b])

    def wait_main(b):
        pltpu.make_async_copy(hjp.at[idxj_v[b]], rows_v[b], sr.at[b]).wait()
        pltpu.make_async_copy(g.at[pl.ds(0, CHUNK)], g_v[b], sg.at[b]).wait()

    def wait_scat(b):
        pltpu.make_async_copy(rows_v[b], acc.at[idxi_v[b]], ss.at[b]).wait()

    def mul(b):
        def mrow(r, c2):
            for k in range(F // 16):
                sl = pl.ds(k * 16, 16)
                rows_v[b][r, sl] = rows_v[b][r, sl] * g_v[b][r, sl]
            return c2
        lax.fori_loop(0, CHUNK, mrow, 0)

    def half(t, b, *, w_scat=True, nxt=True, nxt2=True):
        y = (b + 1) % NBUF
        z = (b + 2) % NBUF
        if nxt:
            wait_idx(y)
            issue_main(t + 1, y)
        if w_scat:
            wait_scat(z)
        if nxt2:
            issue_idx(t + 2, z)
        wait_main(b)
        mul(b)
        pltpu.async_copy(rows_v[b], acc.at[idxi_v[b]], ss.at[b], add=True)

    issue_idx(0, 0)
    issue_idx(1, 1)
    wait_idx(0)
    issue_main(0, 0)
    half(0, 0, w_scat=False)
    half(1, 1, w_scat=False)

    def quad(u, carry):
        t = 2 + u * NBUF
        for r in range(NBUF):
            half(t + r, (2 + r) % NBUF)
        return carry

    lax.fori_loop(0, (NCHUNK - 2) // NBUF - 1, quad, 0)
    tt = 2 + ((NCHUNK - 2) // NBUF - 1) * NBUF
    for r in range(NBUF):
        t = tt + r
        half(t, t % NBUF, nxt=t + 1 < NCHUNK, nxt2=t + 2 < NCHUNK)
    wait_scat((NCHUNK - 2) % NBUF)
    wait_scat((NCHUNK - 1) % NBUF)
    plsc.subcore_barrier()

    pltpu.sync_copy(acc.at[pl.ds(s * RPT, RPT)],
                    out.at[c, pl.ds(s * RPT, RPT)])

    @pl.when(s == NS - 1)
    def _():
        pltpu.sync_copy(acc.at[pl.ds(NS * RPT, TAIL)],
                        out.at[c, pl.ds(NS * RPT, TAIL)])


def _sc_edge(hi0h, hjp, g, pair):
    mesh = plsc.VectorSubcoreMesh(core_axis_name="c", subcore_axis_name="s")
    kern = pl.kernel(
        _sc_edge_body,
        out_type=jax.ShapeDtypeStruct((NC, N, F), jnp.float32),
        mesh=mesh,
        scratch_types=[
            pltpu.VMEM_SHARED((N, F), jnp.float32),
            [pltpu.VMEM((CHUNK,), jnp.int32) for _ in range(NBUF)],
            [pltpu.VMEM((CHUNK,), jnp.int32) for _ in range(NBUF)],
            [pltpu.VMEM((CHUNK, F), jnp.float32) for _ in range(NBUF)],
            [pltpu.VMEM((CHUNK, F), jnp.float32) for _ in range(NBUF)],
            pltpu.SemaphoreType.DMA((NBUF,)),
            pltpu.SemaphoreType.DMA((NBUF,)),
            pltpu.SemaphoreType.DMA((NBUF,)),
            pltpu.SemaphoreType.DMA((NBUF,)),
        ],
    )
    return kern(hi0h, hjp, g, pair.reshape(2, NW, NCHUNK, CHUNK))


def _tail_body(emb_ref, a0_ref, a1_ref,
               iw10, ib10, iw20, ib20, iw11, ib11, iw21, ib21,
               wv, bv_, gate_, ow1, ob1, ow2, ob2, woutp, boutp,
               upd_ref, predp_ref):
    def res(x, w1, b1, w2, b2):
        t = jnp.maximum(x, 0.0)
        u = jnp.maximum(
            jnp.dot(t, w1[...], preferred_element_type=jnp.float32)
            + b1[...], 0.0)
        return x + jnp.dot(u, w2[...], preferred_element_type=jnp.float32) \
            + b2[...]

    h = a0_ref[0] + a1_ref[0]
    h = res(h, iw10, ib10, iw20, ib20)
    h = res(h, iw11, ib11, iw21, ib21)
    upd = gate_[...] * emb_ref[...] + jnp.dot(
        jnp.maximum(h, 0.0), wv[...],
        preferred_element_type=jnp.float32) + bv_[...]
    upd_ref[...] = upd
    y = res(upd, ow1, ob1, ow2, ob2)
    predp_ref[...] = jnp.dot(y, woutp[...],
                             preferred_element_type=jnp.float32) + boutp[...]


def _tail(emb, acc, ir0_W1, ir0_b1, ir0_W2, ir0_b2,
          ir1_W1, ir1_b1, ir1_W2, ir1_b2, Wv, bv, gate,
          or0_W1, or0_b1, or0_W2, or0_b2, Wout, bout):
    woutp = jnp.zeros((F, F), jnp.float32).at[:, :P].set(Wout)
    boutp = jnp.zeros((F,), jnp.float32).at[:P].set(bout)
    grid = (N // NBLK,)
    row = pl.BlockSpec((NBLK, F), lambda i: (i, 0))
    full = pl.BlockSpec((F, F), lambda i: (0, 0))
    vec = pl.BlockSpec((1, F), lambda i: (0, 0))
    a0s = pl.BlockSpec((1, NBLK, F), lambda i: (0, i, 0))
    a1s = pl.BlockSpec((1, NBLK, F), lambda i: (1, i, 0))
    upd, predp = pl.pallas_call(
        _tail_body,
        grid=grid,
        in_specs=[row, a0s, a1s,
                  full, vec, full, vec, full, vec, full, vec,
                  full, vec, vec, full, vec, full, vec, full, vec],
        out_specs=[row, row],
        out_shape=[jax.ShapeDtypeStruct((N, F), jnp.float32),
                   jax.ShapeDtypeStruct((N, F), jnp.float32)],
    )(emb, acc, acc,
      ir0_W1, ir0_b1.reshape(1, F), ir0_W2, ir0_b2.reshape(1, F),
      ir1_W1, ir1_b1.reshape(1, F), ir1_W2, ir1_b2.reshape(1, F),
      Wv, bv.reshape(1, F), gate.reshape(1, F),
      or0_W1, or0_b1.reshape(1, F), or0_W2, or0_b2.reshape(1, F),
      woutp, boutp.reshape(1, F))
    return predp[:, :P], upd


def kernel(atomic_embedding, pair_indices, f_ij, Wi, bi, Wj, bj, Wg, Wv, bv,
           gate, ir0_W1, ir0_b1, ir0_W2, ir0_b2, ir1_W1, ir1_b1, ir1_W2,
           ir1_b2, or0_W1, or0_b1, or0_W2, or0_b2, Wout, bout):
    pair = pair_indices.astype(jnp.int32)
    hi0h, hjp = _node_fwd(atomic_embedding, Wi, bi, Wj, bj)
    g = _edge_gate(f_ij, Wg)
    acc = _sc_edge(hi0h, hjp, g, pair)
    prediction, updated = _tail(
        atomic_embedding, acc,
        ir0_W1, ir0_b1, ir0_W2, ir0_b2, ir1_W1, ir1_b1, ir1_W2, ir1_b2,
        Wv, bv, gate, or0_W1, or0_b1, or0_W2, or0_b2, Wout, bout)
    return (prediction, updated)


# speedup vs baseline: 2.0987x; 1.3394x over previous
import functools
import jax
import jax.numpy as jnp
from jax import lax
from jax.experimental import pallas as pl
from jax.experimental.pallas import tpu as pltpu
from jax.experimental.pallas import tpu_sc as plsc

N, E, F, R, P = 10000, 320000, 128, 16, 2
NC, NS = 2, 16
NW = NC * NS
EPW = E // NW
CHUNK = 40
NCHUNK = EPW // CHUNK
NBUF = 4
RPT = 624
TAIL = N - NS * RPT
NBLK = 1000
EBLK = 8000


def _node_fwd_body(emb_ref, wi_ref, bi_ref, wj_ref, bj_ref, hi0h_ref, hjp_ref):
    emb = emb_ref[...]
    hi = jnp.maximum(
        jnp.dot(emb, wi_ref[...], preferred_element_type=jnp.float32)
        + bi_ref[...], 0.0)
    hi0h_ref[...] = hi * 0.5
    hjp_ref[...] = jnp.maximum(
        jnp.dot(emb, wj_ref[...], preferred_element_type=jnp.float32)
        + bj_ref[...], 0.0)


def _node_fwd(emb, Wi, bi, Wj, bj):
    grid = (N // NBLK,)
    row = pl.BlockSpec((NBLK, F), lambda i: (i, 0))
    full = pl.BlockSpec((F, F), lambda i: (0, 0))
    vec = pl.BlockSpec((1, F), lambda i: (0, 0))
    return pl.pallas_call(
        _node_fwd_body,
        grid=grid,
        in_specs=[row, full, vec, full, vec],
        out_specs=[row, row],
        out_shape=[jax.ShapeDtypeStruct((N, F), jnp.float32),
                   jax.ShapeDtypeStruct((N, F), jnp.float32)],
    )(emb, Wi, bi.reshape(1, F), Wj, bj.reshape(1, F))


def _g_body(f_ref, wg_ref, g_ref):
    g_ref[...] = jnp.dot(f_ref[...], wg_ref[...],
                         preferred_element_type=jnp.float32)


def _edge_gate(f_ij, Wg):
    grid = (E // EBLK,)
    return pl.pallas_call(
        _g_body,
        grid=grid,
        in_specs=[pl.BlockSpec((EBLK, R), lambda i: (i, 0)),
                  pl.BlockSpec((R, F), lambda i: (0, 0))],
        out_specs=pl.BlockSpec((EBLK, F), lambda i: (i, 0)),
        out_shape=jax.ShapeDtypeStruct((E, F), jnp.float32),
    )(f_ij, Wg)


def _sc_edge_body(hi0h, hjp, g, pair, out,
                  acc, idxj_v, idxi_v, rows_v, g_v, si, sr, sg, ss):
    c = lax.axis_index("c")
    s = lax.axis_index("s")
    wid = c * NS + s

    pltpu.sync_copy(hi0h.at[pl.ds(s * RPT, RPT)], acc.at[pl.ds(s * RPT, RPT)])

    @pl.when(s == NS - 1)
    def _():
        pltpu.sync_copy(hi0h.at[pl.ds(NS * RPT, TAIL)],
                        acc.at[pl.ds(NS * RPT, TAIL)])

    plsc.subcore_barrier()

    def issue_idx(t, b):
        pltpu.async_copy(pair.at[1, wid, t], idxj_v[b], si.at[b])
        pltpu.async_copy(pair.at[0, wid, t], idxi_v[b], si.at[b])

    def wait_idx(b):
        pltpu.make_async_copy(pair.at[1, wid, 0], idxj_v[b], si.at[b]).wait()
        pltpu.make_async_copy(pair.at[0, wid, 0], idxi_v[b], si.at[b]).wait()

    def issue_main(t, b):
        pltpu.async_copy(hjp.at[idxj_v[b]], rows_v[b], sr.at[b])
        pltpu.async_copy(g.at[pl.ds((wid * NCHUNK + t) * CHUNK, CHUNK)],
                         g_v[b], sg.at[b])

    def wait_main(b):
        pltpu.make_async_copy(hjp.at[idxj_v[b]], rows_v[b], sr.at[b]).wait()
        pltpu.make_async_copy(g.at[pl.ds(0, CHUNK)], g_v[b], sg.at[b]).wait()

    def wait_scat(b):
        pltpu.make_async_copy(rows_v[b], acc.at[idxi_v[b]], ss.at[b]).wait()

    def mul(b):
        def mrow(r, c2):
            for k in range(F // 16):
                sl = pl.ds(k * 16, 16)
                rows_v[b][r, sl] = rows_v[b][r, sl] * g_v[b][r, sl]
            return c2
        lax.fori_loop(0, CHUNK, mrow, 0)

    def half(t, b, *, w_scat=True, nxt=True, nxt2=True):
        y = (b + 1) % NBUF
        z = (b + 2) % NBUF
        if nxt:
            wait_idx(y)
            issue_main(t + 1, y)
        if w_scat:
            wait_scat(z)
        if nxt2:
            issue_idx(t + 2, z)
        wait_main(b)
        mul(b)
        pltpu.async_copy(rows_v[b], acc.at[idxi_v[b]], ss.at[b], add=True)

    issue_idx(0, 0)
    issue_idx(1, 1)
    wait_idx(0)
    issue_main(0, 0)
    half(0, 0, w_scat=False)
    half(1, 1, w_scat=False)

    def quad(u, carry):
        t = 2 + u * NBUF
        for r in range(NBUF):
            half(t + r, (2 + r) % NBUF)
        return carry

    lax.fori_loop(0, (NCHUNK - 2) // NBUF - 1, quad, 0)
    tt = 2 + ((NCHUNK - 2) // NBUF - 1) * NBUF
    for r in range(NBUF):
        t = tt + r
        half(t, t % NBUF, nxt=t + 1 < NCHUNK, nxt2=t + 2 < NCHUNK)
    wait_scat((NCHUNK - 2) % NBUF)
    wait_scat((NCHUNK - 1) % NBUF)
    plsc.subcore_barrier()

    pltpu.sync_copy(acc.at[pl.ds(s * RPT, RPT)],
                    out.at[c, pl.ds(s * RPT, RPT)])

    @pl.when(s == NS - 1)
    def _():
        pltpu.sync_copy(acc.at[pl.ds(NS * RPT, TAIL)],
                        out.at[c, pl.ds(NS * RPT, TAIL)])


def _sc_edge(hi0h, hjp, g, pair):
    mesh = plsc.VectorSubcoreMesh(core_axis_name="c", subcore_axis_name="s")
    kern = pl.kernel(
        _sc_edge_body,
        out_type=jax.ShapeDtypeStruct((NC, N, F), jnp.float32),
        mesh=mesh,
        scratch_types=[
            pltpu.VMEM_SHARED((N, F), jnp.float32),
            [pltpu.VMEM((CHUNK,), jnp.int32) for _ in range(NBUF)],
            [pltpu.VMEM((CHUNK,), jnp.int32) for _ in range(NBUF)],
            [pltpu.VMEM((CHUNK, F), jnp.float32) for _ in range(NBUF)],
            [pltpu.VMEM((CHUNK, F), jnp.float32) for _ in range(NBUF)],
            pltpu.SemaphoreType.DMA((NBUF,)),
            pltpu.SemaphoreType.DMA((NBUF,)),
            pltpu.SemaphoreType.DMA((NBUF,)),
            pltpu.SemaphoreType.DMA((NBUF,)),
        ],
    )
    return kern(hi0h, hjp, g, pair.reshape(2, NW, NCHUNK, CHUNK))


def _tail_body(emb_ref, a0_ref, a1_ref,
               iw10, ib10, iw20, ib20, iw11, ib11, iw21, ib21,
               wv, bv_, gate_, ow1, ob1, ow2, ob2, woutp, boutp,
               upd_ref, predp_ref):
    def res(x, w1, b1, w2, b2):
        t = jnp.maximum(x, 0.0)
        u = jnp.maximum(
            jnp.dot(t, w1[...], preferred_element_type=jnp.float32)
            + b1[...], 0.0)
        return x + jnp.dot(u, w2[...], preferred_element_type=jnp.float32) \
            + b2[...]

    h = a0_ref[0] + a1_ref[0]
    h = res(h, iw10, ib10, iw20, ib20)
    h = res(h, iw11, ib11, iw21, ib21)
    upd = gate_[...] * emb_ref[...] + jnp.dot(
        jnp.maximum(h, 0.0), wv[...],
        preferred_element_type=jnp.float32) + bv_[...]
    upd_ref[...] = upd
    y = res(upd, ow1, ob1, ow2, ob2)
    predp_ref[...] = jnp.dot(y, woutp[...],
                             preferred_element_type=jnp.float32) + boutp[...]


def _tail(emb, acc, ir0_W1, ir0_b1, ir0_W2, ir0_b2,
          ir1_W1, ir1_b1, ir1_W2, ir1_b2, Wv, bv, gate,
          or0_W1, or0_b1, or0_W2, or0_b2, Wout, bout):
    woutp = jnp.zeros((F, F), jnp.float32).at[:, :P].set(Wout)
    boutp = jnp.zeros((F,), jnp.float32).at[:P].set(bout)
    grid = (N // NBLK,)
    row = pl.BlockSpec((NBLK, F), lambda i: (i, 0))
    full = pl.BlockSpec((F, F), lambda i: (0, 0))
    vec = pl.BlockSpec((1, F), lambda i: (0, 0))
    a0s = pl.BlockSpec((1, NBLK, F), lambda i: (0, i, 0))
    a1s = pl.BlockSpec((1, NBLK, F), lambda i: (1, i, 0))
    upd, predp = pl.pallas_call(
        _tail_body,
        grid=grid,
        in_specs=[row, a0s, a1s,
                  full, vec, full, vec, full, vec, full, vec,
                  full, vec, vec, full, vec, full, vec, full, vec],
        out_specs=[row, row],
        out_shape=[jax.ShapeDtypeStruct((N, F), jnp.float32),
                   jax.ShapeDtypeStruct((N, F), jnp.float32)],
    )(emb, acc, acc,
      ir0_W1, ir0_b1.reshape(1, F), ir0_W2, ir0_b2.reshape(1, F),
      ir1_W1, ir1_b1.reshape(1, F), ir1_W2, ir1_b2.reshape(1, F),
      Wv, bv.reshape(1, F), gate.reshape(1, F),
      or0_W1, or0_b1.reshape(1, F), or0_W2, or0_b2.reshape(1, F),
      woutp, boutp.reshape(1, F))
    return predp[:, :P], upd


def kernel(atomic_embedding, pair_indices, f_ij, Wi, bi, Wj, bj, Wg, Wv, bv,
           gate, ir0_W1, ir0_b1, ir0_W2, ir0_b2, ir1_W1, ir1_b1, ir1_W2,
           ir1_b2, or0_W1, or0_b1, or0_W2, or0_b2, Wout, bout):
    pair = pair_indices.astype(jnp.int32)
    hi0h, hjp = _node_fwd(atomic_embedding, Wi, bi, Wj, bj)
    g = _edge_gate(f_ij, Wg)
    acc = _sc_edge(hi0h, hjp, g, pair)
    prediction, updated = _tail(
        atomic_embedding, acc,
        ir0_W1, ir0_b1, ir0_W2, ir0_b2, ir1_W1, ir1_b1, ir1_W2, ir1_b2,
        Wv, bv, gate, or0_W1, or0_b1, or0_W2, or0_b2, Wout, bout)
    return (prediction, updated)
